# baseline (speedup 1.0000x reference)
import dataclasses
import functools

import jax
import jax.numpy as jnp
from jax import lax
from jax.experimental import pallas as pl
from jax.experimental.pallas import tpu as pltpu
from jax.experimental.pallas import tpu_sc as plsc

N_SUB = 10000
N_GLOBAL = 100000
K = 32
D = 128

NC = 2
NS = 16
NW = NC * NS
N_PAD = 10240
PER_W = N_PAD // NW
CH = 8
NCHUNK = PER_W // CH

_MESH = plsc.VectorSubcoreMesh(core_axis_name="c", subcore_axis_name="s")

_SC_PARAMS = pltpu.CompilerParams(use_tc_tiling_on_sc=False)
if "needs_layout_passes" in pltpu.CompilerParams.__dataclass_fields__:
    _SC_PARAMS = dataclasses.replace(_SC_PARAMS, needs_layout_passes=False)


def _sc_nbr_sum(X_sub, sn_pad, nbr_ids, g2s):
    @functools.partial(
        pl.kernel,
        mesh=_MESH,
        out_type=jax.ShapeDtypeStruct((N_PAD, D), jnp.float32),
        compiler_params=_SC_PARAMS,
        scratch_types=[
            pltpu.VMEM((PER_W,), jnp.int32),
            pltpu.VMEM((PER_W * K,), jnp.int32),
            pltpu.VMEM((PER_W * K,), jnp.int32),
            pltpu.VMEM((PER_W * K,), jnp.int32),
            pltpu.VMEM((CH * K, D), jnp.bfloat16),
            pltpu.VMEM((CH * K, D), jnp.bfloat16),
            pltpu.VMEM((CH, D), jnp.float32),
            pltpu.VMEM((CH, D), jnp.float32),
            pltpu.VMEM_SHARED((N_SUB, D), jnp.bfloat16),
            pltpu.SemaphoreType.DMA,
            pltpu.SemaphoreType.DMA,
            pltpu.SemaphoreType.DMA,
            pltpu.SemaphoreType.DMA,
            pltpu.SemaphoreType.DMA,
            pltpu.SemaphoreType.DMA,
        ],
    )
    def sc_kernel(x_hbm, sn_hbm, nbr_hbm, g2s_hbm, out_hbm,
                  sn_v, nbridx_v, nbrs_v, sidx_v, rows0_v, rows1_v,
                  ost0_v, ost1_v, x_sh,
                  sem_a, sem_g, sem_x0, sem_x1, sem_o0, sem_o1):
        wid = lax.axis_index("s") * NC + lax.axis_index("c")
        wbase = wid * PER_W
        sid = lax.axis_index("s")
        pltpu.sync_copy(x_hbm.at[pl.ds(sid * 624, 624)],
                        x_sh.at[pl.ds(sid * 624, 624)])
        @pl.when(sid == NS - 1)
        def _():
            pltpu.sync_copy(x_hbm.at[pl.ds(9984, 16)],
                            x_sh.at[pl.ds(9984, 16)])
        pltpu.sync_copy(sn_hbm.at[pl.ds(wbase, PER_W)], sn_v)

        kvec0 = lax.iota(jnp.int32, 16) * N_GLOBAL
        kvec1 = kvec0 + 16 * N_GLOBAL

        @pl.loop(0, PER_W)
        def _(i):
            snvec = plsc.load_gather(sn_v, [jnp.full((16,), i, jnp.int32)])
            nbridx_v[pl.ds(i * K, 16)] = kvec0 + snvec
            nbridx_v[pl.ds(i * K + 16, 16)] = kvec1 + snvec

        plsc.subcore_barrier()

        GK = CH * K

        def issue_nbr(c):
            pltpu.async_copy(
                nbr_hbm.at[nbridx_v.at[pl.ds(c * GK, GK)]],
                nbrs_v.at[pl.ds(c * GK, GK)], sem_a)

        def drain_nbr():
            pltpu.make_async_copy(
                nbr_hbm.at[nbridx_v.at[pl.ds(0, GK)]],
                nbrs_v.at[pl.ds(0, GK)], sem_a).wait()

        def issue_g2s(c):
            pltpu.async_copy(
                g2s_hbm.at[nbrs_v.at[pl.ds(c * GK, GK)]],
                sidx_v.at[pl.ds(c * GK, GK)], sem_g)

        def drain_g2s():
            pltpu.make_async_copy(
                g2s_hbm.at[nbrs_v.at[pl.ds(0, GK)]],
                sidx_v.at[pl.ds(0, GK)], sem_g).wait()

        def issue_x(c, rows_ref, sem, src):
            pltpu.async_copy(
                src.at[sidx_v.at[pl.ds(c * CH * K, CH * K)]], rows_ref, sem)

        def wait_x(rows_ref, sem, src):
            pltpu.make_async_copy(
                src.at[sidx_v.at[pl.ds(0, CH * K)]], rows_ref, sem).wait()

        def wait_out(ost_ref, sem):
            pltpu.make_async_copy(
                ost_ref, out_hbm.at[pl.ds(wbase, CH)], sem).wait()

        def reduce_chunk(rows_ref, ost_ref):
            @pl.loop(0, CH)
            def _(n):
                accs = [None] * (D // 32)
                for r4 in range(K // 4):
                    row = n * K + r4 * 4
                    for g in range(D // 32):
                        s01 = (rows_ref.at[row][pl.ds(g * 32, 32)]
                               + rows_ref.at[row + 1][pl.ds(g * 32, 32)])
                        s23 = (rows_ref.at[row + 2][pl.ds(g * 32, 32)]
                               + rows_ref.at[row + 3][pl.ds(g * 32, 32)])
                        a, b = plsc.unpack(
                            s01 + s23, format=plsc.PackFormat.INTERLEAVED)
                        if accs[g] is None:
                            accs[g] = [a, b]
                        else:
                            accs[g][0] = accs[g][0] + a
                            accs[g][1] = accs[g][1] + b
                for g in range(D // 32):
                    ost_ref.at[n][pl.ds(g * 32, 16)] = accs[g][0]
                    ost_ref.at[n][pl.ds(g * 32 + 16, 16)] = accs[g][1]

        def step(c, rows_ref, sem_x, ost_ref, sem_o, src):
            @pl.when(c >= 2)
            def _():
                wait_x(rows_ref, sem_x, src)
                @pl.when(c >= 4)
                def _():
                    wait_out(ost_ref, sem_o)
                reduce_chunk(rows_ref, ost_ref)
            @pl.when(c + 4 < NCHUNK)
            def _():
                issue_nbr(c + 4)
            @pl.when(c + 2 < NCHUNK)
            def _():
                drain_nbr()
                issue_g2s(c + 2)
            @pl.when(c < NCHUNK)
            def _():
                drain_g2s()
                issue_x(c, rows_ref, sem_x, src)
            @pl.when(c >= 2)
            def _():
                pltpu.async_copy(
                    ost_ref, out_hbm.at[pl.ds(wbase + (c - 2) * CH, CH)],
                    sem_o)

        issue_nbr(0)
        issue_nbr(1)
        issue_nbr(2)
        issue_nbr(3)
        drain_nbr()
        issue_g2s(0)
        drain_nbr()
        issue_g2s(1)

        @pl.loop(0, (NCHUNK + 2) // 2)
        def _(t):
            step(2 * t, rows0_v, sem_x0, ost0_v, sem_o0, x_sh)
            step(2 * t + 1, rows1_v, sem_x1, ost1_v, sem_o1, x_hbm)

        wait_out(ost0_v, sem_o0)
        wait_out(ost1_v, sem_o1)

    return sc_kernel(X_sub, sn_pad, nbr_ids, g2s)


def _tc_body(x_ref, s_ref, wn_ref, bn_ref, w1_ref, w2_ref, bf_ref, o_ref):
    mean = s_ref[...] * (1.0 / K)
    h = jnp.dot(mean, wn_ref[...], preferred_element_type=jnp.float32,
                precision=lax.Precision.HIGHEST)
    h = jnp.maximum(h + bn_ref[0:1, :], 0.0)
    o = (jnp.dot(x_ref[...], w1_ref[...], preferred_element_type=jnp.float32,
                 precision=lax.Precision.HIGHEST)
         + jnp.dot(h, w2_ref[...], preferred_element_type=jnp.float32,
                   precision=lax.Precision.HIGHEST)
         + bf_ref[0:1, :])
    o = jnp.maximum(o, 0.0)
    nrm = jnp.sqrt(jnp.sum(o * o, axis=1, keepdims=True))
    o_ref[...] = o / jnp.maximum(nrm, 1e-12)


def _tc_dense(X_sub, nbr_sum, Wn_t, b_nbr, W1_t, W2_t, b_final):
    blk = 2000
    grid = (N_SUB // blk,)
    return pl.pallas_call(
        _tc_body,
        grid=grid,
        in_specs=[
            pl.BlockSpec((blk, D), lambda i: (i, 0)),
            pl.BlockSpec((blk, D), lambda i: (i, 0)),
            pl.BlockSpec((D, D), lambda i: (0, 0)),
            pl.BlockSpec((8, D), lambda i: (0, 0)),
            pl.BlockSpec((D, D), lambda i: (0, 0)),
            pl.BlockSpec((D, D), lambda i: (0, 0)),
            pl.BlockSpec((8, D), lambda i: (0, 0)),
        ],
        out_specs=pl.BlockSpec((blk, D), lambda i: (i, 0)),
        out_shape=jax.ShapeDtypeStruct((N_SUB, D), jnp.float32),
    )(X_sub, nbr_sum, Wn_t, b_nbr, W1_t, W2_t, b_final)


_PERM = []
for _g in range(D // 32):
    _PERM += [_g * 32 + 2 * _j for _j in range(16)]
    _PERM += [_g * 32 + 2 * _j + 1 for _j in range(16)]


def kernel(X_sub, sub_nodes, graphsage_nbr_ids, global_to_sub, W_nbr, b_nbr, W_final, b_final):
    sn_pad = jnp.concatenate(
        [sub_nodes, jnp.zeros((N_PAD - N_SUB,), jnp.int32)])
    X_bf = X_sub.astype(jnp.bfloat16)
    nbr_flat = jnp.ravel(graphsage_nbr_ids.T)
    nbr_sum = _sc_nbr_sum(X_bf, sn_pad, nbr_flat, global_to_sub)
    Wn_t = W_nbr.T[jnp.array(_PERM), :]
    Wf_t = W_final.T
    bn = jnp.broadcast_to(b_nbr[None, :], (8, D))
    bf = jnp.broadcast_to(b_final[None, :], (8, D))
    return _tc_dense(X_sub, nbr_sum, Wn_t, bn,
                     Wf_t[:D], Wf_t[D:], bf)


# speedup vs baseline: 1.0302x; 1.0302x over previous
import dataclasses
import functools

import jax
import jax.numpy as jnp
from jax import lax
from jax.experimental import pallas as pl
from jax.experimental.pallas import tpu as pltpu
from jax.experimental.pallas import tpu_sc as plsc

N_SUB = 10000
N_GLOBAL = 100000
K = 32
D = 128

NC = 2
NS = 16
NW = NC * NS
N_PAD = 10240
PER_W = N_PAD // NW
CH = 8
NCHUNK = PER_W // CH

_MESH = plsc.VectorSubcoreMesh(core_axis_name="c", subcore_axis_name="s")

_SC_PARAMS = pltpu.CompilerParams(use_tc_tiling_on_sc=False)
if "needs_layout_passes" in pltpu.CompilerParams.__dataclass_fields__:
    _SC_PARAMS = dataclasses.replace(_SC_PARAMS, needs_layout_passes=False)


def _sc_nbr_sum(X_sub, sn_pad, nbr_ids, g2s):
    @functools.partial(
        pl.kernel,
        mesh=_MESH,
        out_type=jax.ShapeDtypeStruct((N_PAD, D), jnp.float32),
        compiler_params=_SC_PARAMS,
        scratch_types=[
            pltpu.VMEM((PER_W,), jnp.int32),
            pltpu.VMEM((PER_W * K,), jnp.int32),
            pltpu.VMEM((PER_W * K,), jnp.int32),
            pltpu.VMEM((PER_W * K,), jnp.int32),
            pltpu.VMEM((CH * K, D), jnp.bfloat16),
            pltpu.VMEM((CH * K, D), jnp.bfloat16),
            pltpu.VMEM((CH, D), jnp.float32),
            pltpu.VMEM((CH, D), jnp.float32),
            pltpu.VMEM_SHARED((N_SUB, D), jnp.bfloat16),
            pltpu.SemaphoreType.DMA,
            pltpu.SemaphoreType.DMA,
            pltpu.SemaphoreType.DMA,
            pltpu.SemaphoreType.DMA,
            pltpu.SemaphoreType.DMA,
            pltpu.SemaphoreType.DMA,
        ],
    )
    def sc_kernel(x_hbm, sn_hbm, nbr_hbm, g2s_hbm, out_hbm,
                  sn_v, nbridx_v, nbrs_v, sidx_v, rows0_v, rows1_v,
                  ost0_v, ost1_v, x_sh,
                  sem_a, sem_g, sem_x0, sem_x1, sem_o0, sem_o1):
        wid = lax.axis_index("s") * NC + lax.axis_index("c")
        wbase = wid * PER_W
        sid = lax.axis_index("s")
        pltpu.sync_copy(x_hbm.at[pl.ds(sid * 624, 624)],
                        x_sh.at[pl.ds(sid * 624, 624)])
        @pl.when(sid == NS - 1)
        def _():
            pltpu.sync_copy(x_hbm.at[pl.ds(9984, 16)],
                            x_sh.at[pl.ds(9984, 16)])
        pltpu.sync_copy(sn_hbm.at[pl.ds(wbase, PER_W)], sn_v)

        kvec0 = lax.iota(jnp.int32, 16) * N_GLOBAL
        kvec1 = kvec0 + 16 * N_GLOBAL

        @pl.loop(0, PER_W)
        def _(i):
            snvec = plsc.load_gather(sn_v, [jnp.full((16,), i, jnp.int32)])
            nbridx_v[pl.ds(i * K, 16)] = kvec0 + snvec
            nbridx_v[pl.ds(i * K + 16, 16)] = kvec1 + snvec

        plsc.subcore_barrier()

        GK = CH * K

        def issue_nbr(c):
            pltpu.async_copy(
                nbr_hbm.at[nbridx_v.at[pl.ds(c * GK, GK)]],
                nbrs_v.at[pl.ds(c * GK, GK)], sem_a)

        def drain_nbr():
            pltpu.make_async_copy(
                nbr_hbm.at[nbridx_v.at[pl.ds(0, GK)]],
                nbrs_v.at[pl.ds(0, GK)], sem_a).wait()

        def issue_g2s(c):
            pltpu.async_copy(
                g2s_hbm.at[nbrs_v.at[pl.ds(c * GK, GK)]],
                sidx_v.at[pl.ds(c * GK, GK)], sem_g)

        def drain_g2s():
            pltpu.make_async_copy(
                g2s_hbm.at[nbrs_v.at[pl.ds(0, GK)]],
                sidx_v.at[pl.ds(0, GK)], sem_g).wait()

        def issue_x(c, rows_ref, sem):
            pltpu.async_copy(
                x_sh.at[sidx_v.at[pl.ds(c * CH * K, CH * K)]], rows_ref, sem)

        def wait_x(rows_ref, sem):
            pltpu.make_async_copy(
                x_sh.at[sidx_v.at[pl.ds(0, CH * K)]], rows_ref, sem).wait()

        def wait_out(ost_ref, sem):
            pltpu.make_async_copy(
                ost_ref, out_hbm.at[pl.ds(wbase, CH)], sem).wait()

        def reduce_chunk(rows_ref, ost_ref):
            @pl.loop(0, CH)
            def _(n):
                accs = [None] * (D // 32)
                for r4 in range(K // 4):
                    row = n * K + r4 * 4
                    for g in range(D // 32):
                        s01 = (rows_ref.at[row][pl.ds(g * 32, 32)]
                               + rows_ref.at[row + 1][pl.ds(g * 32, 32)])
                        s23 = (rows_ref.at[row + 2][pl.ds(g * 32, 32)]
                               + rows_ref.at[row + 3][pl.ds(g * 32, 32)])
                        a, b = plsc.unpack(
                            s01 + s23, format=plsc.PackFormat.INTERLEAVED)
                        if accs[g] is None:
                            accs[g] = [a, b]
                        else:
                            accs[g][0] = accs[g][0] + a
                            accs[g][1] = accs[g][1] + b
                for g in range(D // 32):
                    ost_ref.at[n][pl.ds(g * 32, 16)] = accs[g][0]
                    ost_ref.at[n][pl.ds(g * 32 + 16, 16)] = accs[g][1]

        def step(c, rows_ref, sem_x, ost_ref, sem_o):
            @pl.when(c >= 2)
            def _():
                wait_x(rows_ref, sem_x)
                @pl.when(c >= 4)
                def _():
                    wait_out(ost_ref, sem_o)
                reduce_chunk(rows_ref, ost_ref)
            @pl.when(c + 4 < NCHUNK)
            def _():
                issue_nbr(c + 4)
            @pl.when(c + 2 < NCHUNK)
            def _():
                drain_nbr()
                issue_g2s(c + 2)
            @pl.when(c < NCHUNK)
            def _():
                drain_g2s()
                issue_x(c, rows_ref, sem_x)
            @pl.when(c >= 2)
            def _():
                pltpu.async_copy(
                    ost_ref, out_hbm.at[pl.ds(wbase + (c - 2) * CH, CH)],
                    sem_o)

        issue_nbr(0)
        issue_nbr(1)
        issue_nbr(2)
        issue_nbr(3)
        drain_nbr()
        issue_g2s(0)
        drain_nbr()
        issue_g2s(1)

        @pl.loop(0, (NCHUNK + 2) // 2)
        def _(t):
            step(2 * t, rows0_v, sem_x0, ost0_v, sem_o0)
            step(2 * t + 1, rows1_v, sem_x1, ost1_v, sem_o1)

        wait_out(ost0_v, sem_o0)
        wait_out(ost1_v, sem_o1)

    return sc_kernel(X_sub, sn_pad, nbr_ids, g2s)


def _tc1_body(x_ref, w1_ref, p_ref):
    p_ref[...] = jnp.dot(x_ref[...], w1_ref[...],
                         preferred_element_type=jnp.float32,
                         precision=lax.Precision.HIGHEST)


def _tc1_xw(X_sub, W1_t):
    blk = 2000
    return pl.pallas_call(
        _tc1_body,
        grid=(N_SUB // blk,),
        in_specs=[
            pl.BlockSpec((blk, D), lambda i: (i, 0)),
            pl.BlockSpec((D, D), lambda i: (0, 0)),
        ],
        out_specs=pl.BlockSpec((blk, D), lambda i: (i, 0)),
        out_shape=jax.ShapeDtypeStruct((N_SUB, D), jnp.float32),
    )(X_sub, W1_t)


def _tc2_body(p_ref, s_ref, wn_ref, bn_ref, w2_ref, bf_ref, o_ref):
    mean = s_ref[...] * (1.0 / K)
    h = jnp.dot(mean, wn_ref[...], preferred_element_type=jnp.float32,
                precision=lax.Precision.HIGHEST)
    h = jnp.maximum(h + bn_ref[0:1, :], 0.0)
    o = (p_ref[...]
         + jnp.dot(h, w2_ref[...], preferred_element_type=jnp.float32,
                   precision=lax.Precision.HIGHEST)
         + bf_ref[0:1, :])
    o = jnp.maximum(o, 0.0)
    nrm = jnp.sqrt(jnp.sum(o * o, axis=1, keepdims=True))
    o_ref[...] = o / jnp.maximum(nrm, 1e-12)


def _tc_dense(P, nbr_sum, Wn_t, b_nbr, W2_t, b_final):
    blk = 2000
    grid = (N_SUB // blk,)
    return pl.pallas_call(
        _tc2_body,
        grid=grid,
        in_specs=[
            pl.BlockSpec((blk, D), lambda i: (i, 0)),
            pl.BlockSpec((blk, D), lambda i: (i, 0)),
            pl.BlockSpec((D, D), lambda i: (0, 0)),
            pl.BlockSpec((8, D), lambda i: (0, 0)),
            pl.BlockSpec((D, D), lambda i: (0, 0)),
            pl.BlockSpec((8, D), lambda i: (0, 0)),
        ],
        out_specs=pl.BlockSpec((blk, D), lambda i: (i, 0)),
        out_shape=jax.ShapeDtypeStruct((N_SUB, D), jnp.float32),
    )(P, nbr_sum, Wn_t, b_nbr, W2_t, b_final)


_PERM = []
for _g in range(D // 32):
    _PERM += [_g * 32 + 2 * _j for _j in range(16)]
    _PERM += [_g * 32 + 2 * _j + 1 for _j in range(16)]


def kernel(X_sub, sub_nodes, graphsage_nbr_ids, global_to_sub, W_nbr, b_nbr, W_final, b_final):
    sn_pad = jnp.concatenate(
        [sub_nodes, jnp.zeros((N_PAD - N_SUB,), jnp.int32)])
    X_bf = X_sub.astype(jnp.bfloat16)
    nbr_flat = jnp.ravel(graphsage_nbr_ids.T)
    nbr_sum = _sc_nbr_sum(X_bf, sn_pad, nbr_flat, global_to_sub)
    Wn_t = W_nbr.T[jnp.array(_PERM), :]
    Wf_t = W_final.T
    bn = jnp.broadcast_to(b_nbr[None, :], (8, D))
    bf = jnp.broadcast_to(b_final[None, :], (8, D))
    P = _tc1_xw(X_sub, Wf_t[:D])
    return _tc_dense(P, nbr_sum, Wn_t, bn, Wf_t[D:], bf)


# speedup vs baseline: 1.0697x; 1.0384x over previous
import dataclasses
import functools

import jax
import jax.numpy as jnp
from jax import lax
from jax.experimental import pallas as pl
from jax.experimental.pallas import tpu as pltpu
from jax.experimental.pallas import tpu_sc as plsc

N_SUB = 10000
N_GLOBAL = 100000
K = 32
D = 128

NC = 2
NS = 16
NW = NC * NS
N_PAD = 10240
PER_W = N_PAD // NW
CH = 8
NCHUNK = PER_W // CH

_MESH = plsc.VectorSubcoreMesh(core_axis_name="c", subcore_axis_name="s")

_SC_PARAMS = pltpu.CompilerParams(use_tc_tiling_on_sc=False)
if "needs_layout_passes" in pltpu.CompilerParams.__dataclass_fields__:
    _SC_PARAMS = dataclasses.replace(_SC_PARAMS, needs_layout_passes=False)


def _sc_nbr_sum(X_sub, sn_pad, nbr_ids, g2s):
    @functools.partial(
        pl.kernel,
        mesh=_MESH,
        out_type=jax.ShapeDtypeStruct((N_PAD, D), jnp.float32),
        compiler_params=_SC_PARAMS,
        scratch_types=[
            pltpu.VMEM((PER_W,), jnp.int32),
            pltpu.VMEM((PER_W * K,), jnp.int32),
            pltpu.VMEM((PER_W * K,), jnp.int32),
            pltpu.VMEM((PER_W * K,), jnp.int32),
            pltpu.VMEM((CH * K, D), jnp.bfloat16),
            pltpu.VMEM((CH * K, D), jnp.bfloat16),
            pltpu.VMEM((CH, D), jnp.float32),
            pltpu.VMEM((CH, D), jnp.float32),
            pltpu.VMEM_SHARED((N_SUB, D), jnp.bfloat16),
            pltpu.VMEM_SHARED((N_GLOBAL,), jnp.int32),
            pltpu.SemaphoreType.DMA,
            pltpu.SemaphoreType.DMA,
            pltpu.SemaphoreType.DMA,
            pltpu.SemaphoreType.DMA,
            pltpu.SemaphoreType.DMA,
            pltpu.SemaphoreType.DMA,
        ],
    )
    def sc_kernel(x_hbm, sn_hbm, nbr_hbm, g2s_hbm, out_hbm,
                  sn_v, nbridx_v, nbrs_v, sidx_v, rows0_v, rows1_v,
                  ost0_v, ost1_v, x_sh, g2s_sh,
                  sem_a, sem_g, sem_x0, sem_x1, sem_o0, sem_o1):
        wid = lax.axis_index("s") * NC + lax.axis_index("c")
        wbase = wid * PER_W
        sid = lax.axis_index("s")
        pltpu.sync_copy(x_hbm.at[pl.ds(sid * 624, 624)],
                        x_sh.at[pl.ds(sid * 624, 624)])
        pltpu.sync_copy(g2s_hbm.at[pl.ds(sid * 6248, 6248)],
                        g2s_sh.at[pl.ds(sid * 6248, 6248)])
        @pl.when(sid == NS - 1)
        def _():
            pltpu.sync_copy(x_hbm.at[pl.ds(9984, 16)],
                            x_sh.at[pl.ds(9984, 16)])
            pltpu.sync_copy(g2s_hbm.at[pl.ds(99968, 32)],
                            g2s_sh.at[pl.ds(99968, 32)])
        pltpu.sync_copy(sn_hbm.at[pl.ds(wbase, PER_W)], sn_v)

        kvec0 = lax.iota(jnp.int32, 16) * N_GLOBAL
        kvec1 = kvec0 + 16 * N_GLOBAL

        @pl.loop(0, PER_W)
        def _(i):
            snvec = plsc.load_gather(sn_v, [jnp.full((16,), i, jnp.int32)])
            nbridx_v[pl.ds(i * K, 16)] = kvec0 + snvec
            nbridx_v[pl.ds(i * K + 16, 16)] = kvec1 + snvec

        plsc.subcore_barrier()

        GK = CH * K

        def issue_nbr(c):
            pltpu.async_copy(
                nbr_hbm.at[nbridx_v.at[pl.ds(c * GK, GK)]],
                nbrs_v.at[pl.ds(c * GK, GK)], sem_a)

        def drain_nbr():
            pltpu.make_async_copy(
                nbr_hbm.at[nbridx_v.at[pl.ds(0, GK)]],
                nbrs_v.at[pl.ds(0, GK)], sem_a).wait()

        def issue_g2s(c):
            pltpu.async_copy(
                g2s_sh.at[nbrs_v.at[pl.ds(c * GK, GK)]],
                sidx_v.at[pl.ds(c * GK, GK)], sem_g)

        def drain_g2s():
            pltpu.make_async_copy(
                g2s_sh.at[nbrs_v.at[pl.ds(0, GK)]],
                sidx_v.at[pl.ds(0, GK)], sem_g).wait()

        def issue_x(c, rows_ref, sem):
            pltpu.async_copy(
                x_sh.at[sidx_v.at[pl.ds(c * CH * K, CH * K)]], rows_ref, sem)

        def wait_x(rows_ref, sem):
            pltpu.make_async_copy(
                x_sh.at[sidx_v.at[pl.ds(0, CH * K)]], rows_ref, sem).wait()

        def wait_out(ost_ref, sem):
            pltpu.make_async_copy(
                ost_ref, out_hbm.at[pl.ds(wbase, CH)], sem).wait()

        def reduce_chunk(rows_ref, ost_ref):
            @pl.loop(0, CH)
            def _(n):
                accs = [None] * (D // 32)
                for r4 in range(K // 4):
                    row = n * K + r4 * 4
                    for g in range(D // 32):
                        s01 = (rows_ref.at[row][pl.ds(g * 32, 32)]
                               + rows_ref.at[row + 1][pl.ds(g * 32, 32)])
                        s23 = (rows_ref.at[row + 2][pl.ds(g * 32, 32)]
                               + rows_ref.at[row + 3][pl.ds(g * 32, 32)])
                        a, b = plsc.unpack(
                            s01 + s23, format=plsc.PackFormat.INTERLEAVED)
                        if accs[g] is None:
                            accs[g] = [a, b]
                        else:
                            accs[g][0] = accs[g][0] + a
                            accs[g][1] = accs[g][1] + b
                for g in range(D // 32):
                    ost_ref.at[n][pl.ds(g * 32, 16)] = accs[g][0]
                    ost_ref.at[n][pl.ds(g * 32 + 16, 16)] = accs[g][1]

        def step(c, rows_ref, sem_x, ost_ref, sem_o):
            @pl.when(c >= 2)
            def _():
                wait_x(rows_ref, sem_x)
                @pl.when(c >= 4)
                def _():
                    wait_out(ost_ref, sem_o)
                reduce_chunk(rows_ref, ost_ref)
            @pl.when(c + 4 < NCHUNK)
            def _():
                issue_nbr(c + 4)
            @pl.when(c + 2 < NCHUNK)
            def _():
                drain_nbr()
                issue_g2s(c + 2)
            @pl.when(c < NCHUNK)
            def _():
                drain_g2s()
                issue_x(c, rows_ref, sem_x)
            @pl.when(c >= 2)
            def _():
                pltpu.async_copy(
                    ost_ref, out_hbm.at[pl.ds(wbase + (c - 2) * CH, CH)],
                    sem_o)

        issue_nbr(0)
        issue_nbr(1)
        issue_nbr(2)
        issue_nbr(3)
        drain_nbr()
        issue_g2s(0)
        drain_nbr()
        issue_g2s(1)

        @pl.loop(0, (NCHUNK + 2) // 2)
        def _(t):
            step(2 * t, rows0_v, sem_x0, ost0_v, sem_o0)
            step(2 * t + 1, rows1_v, sem_x1, ost1_v, sem_o1)

        wait_out(ost0_v, sem_o0)
        wait_out(ost1_v, sem_o1)

    return sc_kernel(X_sub, sn_pad, nbr_ids, g2s)


def _tc1_body(x_ref, w1_ref, p_ref):
    p_ref[...] = jnp.dot(x_ref[...], w1_ref[...],
                         preferred_element_type=jnp.float32,
                         precision=lax.Precision.HIGHEST)


def _tc1_xw(X_sub, W1_t):
    blk = 2000
    return pl.pallas_call(
        _tc1_body,
        grid=(N_SUB // blk,),
        in_specs=[
            pl.BlockSpec((blk, D), lambda i: (i, 0)),
            pl.BlockSpec((D, D), lambda i: (0, 0)),
        ],
        out_specs=pl.BlockSpec((blk, D), lambda i: (i, 0)),
        out_shape=jax.ShapeDtypeStruct((N_SUB, D), jnp.float32),
    )(X_sub, W1_t)


def _tc2_body(p_ref, s_ref, wn_ref, bn_ref, w2_ref, bf_ref, o_ref):
    mean = s_ref[...] * (1.0 / K)
    h = jnp.dot(mean, wn_ref[...], preferred_element_type=jnp.float32,
                precision=lax.Precision.HIGHEST)
    h = jnp.maximum(h + bn_ref[0:1, :], 0.0)
    o = (p_ref[...]
         + jnp.dot(h, w2_ref[...], preferred_element_type=jnp.float32,
                   precision=lax.Precision.HIGHEST)
         + bf_ref[0:1, :])
    o = jnp.maximum(o, 0.0)
    nrm = jnp.sqrt(jnp.sum(o * o, axis=1, keepdims=True))
    o_ref[...] = o / jnp.maximum(nrm, 1e-12)


def _tc_dense(P, nbr_sum, Wn_t, b_nbr, W2_t, b_final):
    blk = 2000
    grid = (N_SUB // blk,)
    return pl.pallas_call(
        _tc2_body,
        grid=grid,
        in_specs=[
            pl.BlockSpec((blk, D), lambda i: (i, 0)),
            pl.BlockSpec((blk, D), lambda i: (i, 0)),
            pl.BlockSpec((D, D), lambda i: (0, 0)),
            pl.BlockSpec((8, D), lambda i: (0, 0)),
            pl.BlockSpec((D, D), lambda i: (0, 0)),
            pl.BlockSpec((8, D), lambda i: (0, 0)),
        ],
        out_specs=pl.BlockSpec((blk, D), lambda i: (i, 0)),
        out_shape=jax.ShapeDtypeStruct((N_SUB, D), jnp.float32),
    )(P, nbr_sum, Wn_t, b_nbr, W2_t, b_final)


_PERM = []
for _g in range(D // 32):
    _PERM += [_g * 32 + 2 * _j for _j in range(16)]
    _PERM += [_g * 32 + 2 * _j + 1 for _j in range(16)]


def kernel(X_sub, sub_nodes, graphsage_nbr_ids, global_to_sub, W_nbr, b_nbr, W_final, b_final):
    sn_pad = jnp.concatenate(
        [sub_nodes, jnp.zeros((N_PAD - N_SUB,), jnp.int32)])
    X_bf = X_sub.astype(jnp.bfloat16)
    nbr_flat = jnp.ravel(graphsage_nbr_ids.T)
    nbr_sum = _sc_nbr_sum(X_bf, sn_pad, nbr_flat, global_to_sub)
    Wn_t = W_nbr.T[jnp.array(_PERM), :]
    Wf_t = W_final.T
    bn = jnp.broadcast_to(b_nbr[None, :], (8, D))
    bf = jnp.broadcast_to(b_final[None, :], (8, D))
    P = _tc1_xw(X_sub, Wf_t[:D])
    return _tc_dense(P, nbr_sum, Wn_t, bn, Wf_t[D:], bf)
